# baseline (speedup 1.0000x reference)
import functools

import jax
import jax.numpy as jnp
from jax import lax
from jax.experimental import pallas as pl
from jax.experimental.pallas import tpu as pltpu
from jax.experimental.pallas import tpu_sc as plsc

N_SP = 1024
NC = 2
NS = 16
LANES = 16
HROWS = 8
NBUF = 2


@functools.partial(jax.jit, static_argnums=(4, 5, 6, 7, 8))
def _superpixel_pool(x, labels, e0, e1, B, C, H, W, NE):
    NW = NC * NS
    WPI = NW // B
    CPB = C // WPI
    n_chunks = H // HROWS
    wgroups = W // LANES
    acc_len = CPB * N_SP

    mesh = plsc.VectorSubcoreMesh(
        core_axis_name="c", subcore_axis_name="s",
        num_cores=NC, num_subcores=NS)

    @functools.partial(
        pl.kernel,
        out_type=(
            jax.ShapeDtypeStruct((NW * CPB * NE,), jnp.float32),
            jax.ShapeDtypeStruct((NW * CPB * NE,), jnp.float32),
        ),
        mesh=mesh,
        compiler_params=pltpu.CompilerParams(needs_layout_passes=False),
        scratch_types=[
            pltpu.VMEM((NBUF, HROWS, W), jnp.int32),
            pltpu.VMEM((NBUF, CPB, HROWS, W), jnp.float32),
            pltpu.SemaphoreType.DMA((NBUF,)),
            [pltpu.VMEM((N_SP,), jnp.float32)] * CPB,
            pltpu.VMEM((N_SP,), jnp.float32),
            pltpu.VMEM((N_SP,), jnp.float32),
            pltpu.VMEM((1, NE), jnp.int32),
            pltpu.VMEM((1, NE), jnp.int32),
            pltpu.VMEM((CPB * NE,), jnp.float32),
            pltpu.VMEM((CPB * NE,), jnp.float32),
        ],
    )
    def sp_kernel(x_hbm, lbl_hbm, e0_hbm, e1_hbm, out0_hbm, out1_hbm,
                  lblbuf, xbuf, sem, accum, cnt, rcp, e0buf, e1buf, ob0, ob1):
        wid = lax.axis_index("s") * NC + lax.axis_index("c")
        b = wid // WPI
        cb = wid % WPI
        c0 = cb * CPB

        zeros = jnp.zeros((LANES,), jnp.float32)
        ones = jnp.ones((LANES,), jnp.float32)

        def zero_acc(i, _):
            for c in range(CPB):
                accum[c][pl.ds(i * LANES, LANES)] = zeros
            return 0
        lax.fori_loop(0, N_SP // LANES, zero_acc, 0)

        def zero_cnt(i, _):
            cnt[pl.ds(i * LANES, LANES)] = zeros
            return 0
        lax.fori_loop(0, N_SP // LANES, zero_cnt, 0)

        def copies(j, slot):
            h0 = j * HROWS
            return (
                pltpu.make_async_copy(
                    lbl_hbm.at[b, 0, pl.ds(h0, HROWS), :],
                    lblbuf.at[slot], sem.at[slot]),
                pltpu.make_async_copy(
                    x_hbm.at[b, pl.ds(c0, CPB), pl.ds(h0, HROWS), :],
                    xbuf.at[slot], sem.at[slot]),
            )

        def issue(j, slot):
            for cp in copies(j, slot):
                cp.start()

        def drain(j, slot):
            for cp in copies(j, slot):
                cp.wait()

        issue(0, 0)

        def chunk_body(j, _):
            slot = lax.rem(j, NBUF)

            @pl.when(j + 1 < n_chunks)
            def _():
                issue(j + 1, lax.rem(j + 1, NBUF))

            drain(j, slot)

            @plsc.parallel_loop(0, HROWS * wgroups, 1, unroll=2)
            def group_body(g):
                r = g // wgroups
                p = (g % wgroups) * LANES
                lbl = lblbuf[slot, r, pl.ds(p, LANES)]
                plsc.addupdate_scatter(cnt, [lbl], ones)
                for c in range(CPB):
                    v = xbuf[slot, c, r, pl.ds(p, LANES)]
                    plsc.addupdate_scatter(accum[c], [lbl], v)
            return 0
        lax.fori_loop(0, n_chunks, chunk_body, 0)

        def recip_body(i, _):
            s = pl.ds(i * LANES, LANES)
            rcp[s] = 1.0 / jnp.maximum(cnt[s], 1.0)
            return 0
        lax.fori_loop(0, N_SP // LANES, recip_body, 0)

        def mean_body(i, _):
            s = pl.ds(i * LANES, LANES)
            r = rcp[s]
            for c in range(CPB):
                accum[c][s] = accum[c][s] * r
            return 0
        lax.fori_loop(0, N_SP // LANES, mean_body, 0)

        pltpu.sync_copy(e0_hbm.at[b], e0buf)
        pltpu.sync_copy(e1_hbm.at[b], e1buf)
        for g in range(NE // LANES):
            p = g * LANES
            i0 = e0buf[0, pl.ds(p, LANES)]
            i1 = e1buf[0, pl.ds(p, LANES)]
            for c in range(CPB):
                ob0[pl.ds(c * NE + p, LANES)] = plsc.load_gather(
                    accum[c], [i0])
                ob1[pl.ds(c * NE + p, LANES)] = plsc.load_gather(
                    accum[c], [i1])
        pltpu.sync_copy(ob0, out0_hbm.at[pl.ds(wid * CPB * NE, CPB * NE)])
        pltpu.sync_copy(ob1, out1_hbm.at[pl.ds(wid * CPB * NE, CPB * NE)])

    return sp_kernel(x, labels, e0, e1)


def kernel(x, graphs, label_maps, edges_to_pool):
    B, C, H, W = x.shape
    NE = edges_to_pool.shape[1]
    e0 = edges_to_pool[:, :, 0].reshape(B, 1, NE)
    e1 = edges_to_pool[:, :, 1].reshape(B, 1, NE)
    out0, out1 = _superpixel_pool(x, label_maps, e0, e1, B, C, H, W, NE)
    NW = NC * NS
    WPI = NW // B
    CPB = C // WPI
    X0 = out0.reshape(B, WPI, CPB, NE).transpose(0, 3, 1, 2).reshape(B, NE, C)
    X1 = out1.reshape(B, WPI, CPB, NE).transpose(0, 3, 1, 2).reshape(B, NE, C)
    Y = edges_to_pool[:, :, 2].astype(x.dtype)[:, :, None]
    return X0, X1, Y


# speedup vs baseline: 1.0001x; 1.0001x over previous
import functools

import jax
import jax.numpy as jnp
from jax import lax
from jax.experimental import pallas as pl
from jax.experimental.pallas import tpu as pltpu
from jax.experimental.pallas import tpu_sc as plsc

N_SP = 1024
NC = 2
NS = 16
LANES = 16
HROWS = 8
NBUF = 2


@functools.partial(jax.jit, static_argnums=(4, 5, 6, 7, 8))
def _superpixel_pool(x, labels, e0, e1, B, C, H, W, NE):
    NW = NC * NS
    WPI = NW // B
    CPB = C // WPI
    n_chunks = H // HROWS
    wgroups = W // LANES
    acc_len = CPB * N_SP

    mesh = plsc.VectorSubcoreMesh(
        core_axis_name="c", subcore_axis_name="s",
        num_cores=NC, num_subcores=NS)

    @functools.partial(
        pl.kernel,
        out_type=(
            jax.ShapeDtypeStruct((NW * CPB * NE,), jnp.float32),
            jax.ShapeDtypeStruct((NW * CPB * NE,), jnp.float32),
        ),
        mesh=mesh,
        compiler_params=pltpu.CompilerParams(needs_layout_passes=False),
        scratch_types=[
            pltpu.VMEM((NBUF, HROWS, W), jnp.int32),
            pltpu.VMEM((NBUF, CPB, HROWS, W), jnp.float32),
            pltpu.SemaphoreType.DMA((NBUF,)),
            [pltpu.VMEM((N_SP,), jnp.float32)] * CPB,
            pltpu.VMEM((N_SP,), jnp.float32),
            pltpu.VMEM((N_SP,), jnp.float32),
            pltpu.VMEM((1, NE), jnp.int32),
            pltpu.VMEM((1, NE), jnp.int32),
            pltpu.VMEM((CPB * NE,), jnp.float32),
            pltpu.VMEM((CPB * NE,), jnp.float32),
        ],
    )
    def sp_kernel(x_hbm, lbl_hbm, e0_hbm, e1_hbm, out0_hbm, out1_hbm,
                  lblbuf, xbuf, sem, accum, cnt, rcp, e0buf, e1buf, ob0, ob1):
        wid = lax.axis_index("s") * NC + lax.axis_index("c")
        b = wid // WPI
        cb = wid % WPI
        c0 = cb * CPB

        zeros = jnp.zeros((LANES,), jnp.float32)
        ones = jnp.ones((LANES,), jnp.float32)

        def zero_acc(i, _):
            for c in range(CPB):
                accum[c][pl.ds(i * LANES, LANES)] = zeros
            return 0
        lax.fori_loop(0, N_SP // LANES, zero_acc, 0)

        def zero_cnt(i, _):
            cnt[pl.ds(i * LANES, LANES)] = zeros
            return 0
        lax.fori_loop(0, N_SP // LANES, zero_cnt, 0)

        def copies(j, slot):
            h0 = j * HROWS
            return (
                pltpu.make_async_copy(
                    lbl_hbm.at[b, 0, pl.ds(h0, HROWS), :],
                    lblbuf.at[slot], sem.at[slot]),
                pltpu.make_async_copy(
                    x_hbm.at[b, pl.ds(c0, CPB), pl.ds(h0, HROWS), :],
                    xbuf.at[slot], sem.at[slot]),
            )

        def issue(j, slot):
            for cp in copies(j, slot):
                cp.start()

        def drain(j, slot):
            for cp in copies(j, slot):
                cp.wait()

        issue(0, 0)

        def chunk_body(j, _):
            slot = lax.rem(j, NBUF)

            @pl.when(j + 1 < n_chunks)
            def _():
                issue(j + 1, lax.rem(j + 1, NBUF))

            drain(j, slot)

            @plsc.parallel_loop(0, HROWS * wgroups, 1, unroll=1)
            def group_body(g):
                r = g // wgroups
                p = (g % wgroups) * LANES
                lbl = lblbuf[slot, r, pl.ds(p, LANES)]
                plsc.addupdate_scatter(cnt, [lbl], ones)
                for c in range(CPB):
                    v = xbuf[slot, c, r, pl.ds(p, LANES)]
                    plsc.addupdate_scatter(accum[c], [lbl], v)
            return 0
        lax.fori_loop(0, n_chunks, chunk_body, 0)

        def recip_body(i, _):
            s = pl.ds(i * LANES, LANES)
            rcp[s] = 1.0 / jnp.maximum(cnt[s], 1.0)
            return 0
        lax.fori_loop(0, N_SP // LANES, recip_body, 0)

        def mean_body(i, _):
            s = pl.ds(i * LANES, LANES)
            r = rcp[s]
            for c in range(CPB):
                accum[c][s] = accum[c][s] * r
            return 0
        lax.fori_loop(0, N_SP // LANES, mean_body, 0)

        pltpu.sync_copy(e0_hbm.at[b], e0buf)
        pltpu.sync_copy(e1_hbm.at[b], e1buf)
        for g in range(NE // LANES):
            p = g * LANES
            i0 = e0buf[0, pl.ds(p, LANES)]
            i1 = e1buf[0, pl.ds(p, LANES)]
            for c in range(CPB):
                ob0[pl.ds(c * NE + p, LANES)] = plsc.load_gather(
                    accum[c], [i0])
                ob1[pl.ds(c * NE + p, LANES)] = plsc.load_gather(
                    accum[c], [i1])
        pltpu.sync_copy(ob0, out0_hbm.at[pl.ds(wid * CPB * NE, CPB * NE)])
        pltpu.sync_copy(ob1, out1_hbm.at[pl.ds(wid * CPB * NE, CPB * NE)])

    return sp_kernel(x, labels, e0, e1)


def kernel(x, graphs, label_maps, edges_to_pool):
    B, C, H, W = x.shape
    NE = edges_to_pool.shape[1]
    e0 = edges_to_pool[:, :, 0].reshape(B, 1, NE)
    e1 = edges_to_pool[:, :, 1].reshape(B, 1, NE)
    out0, out1 = _superpixel_pool(x, label_maps, e0, e1, B, C, H, W, NE)
    NW = NC * NS
    WPI = NW // B
    CPB = C // WPI
    X0 = out0.reshape(B, WPI, CPB, NE).transpose(0, 3, 1, 2).reshape(B, NE, C)
    X1 = out1.reshape(B, WPI, CPB, NE).transpose(0, 3, 1, 2).reshape(B, NE, C)
    Y = edges_to_pool[:, :, 2].astype(x.dtype)[:, :, None]
    return X0, X1, Y
